# baseline (speedup 1.0000x reference)
import functools
import math

import jax
import jax.numpy as jnp
from jax import lax
from jax.experimental import pallas as pl
from jax.experimental.pallas import tpu as pltpu
from jax.experimental.pallas import tpu_sc as plsc

_SMOOTH = 0.1
_VOCAB = 32000
_EPS = _SMOOTH / (_VOCAB - 1)
_CONF = 1.0 - _SMOOTH
_CD = _CONF - _EPS
_C = (_VOCAB - 1) * _EPS * math.log(_EPS) + _CONF * math.log(_CONF)

_NC, _NS, _L = 2, 16, 16
_NW = _NC * _NS
_SC_COLS = 6400
_TC_COLS = _VOCAB - _SC_COLS


def _tc_body(pred_ref, tgt_ref, s_ref, n_ref):
    i = pl.program_id(0)
    m = (tgt_ref[...] > 0).astype(jnp.float32)
    rs = jnp.sum(pred_ref[...], axis=1, keepdims=True)
    part = jnp.sum(rs * m)

    @pl.when(i == 0)
    def _():
        s_ref[...] = jnp.zeros_like(s_ref)
        n_ref[...] = jnp.sum(m).reshape(1, 1)

    s_ref[...] += part.reshape(1, 1)


def _sc_body(bpw, pred_hbm, tgt_hbm, out_hbm,
             tgt_v, chunk_v, row_v, acc_v, sem, sem_a, sem_b):
    wid = lax.axis_index("s") * _NC + lax.axis_index("c")
    base = wid * bpw
    pltpu.sync_copy(tgt_hbm.at[pl.ds(base, bpw)], tgt_v.at[pl.ds(0, bpw)])
    copies = []
    for j in range(bpw):
        t = tgt_v[pl.ds(j, _L)][0]
        cc = (t // _L) * _L
        copies.append(pltpu.async_copy(
            pred_hbm.at[base + j, pl.ds(cc, _L)],
            chunk_v.at[pl.ds(j * _L, _L)], sem))
    row_sems = (sem_a, sem_b)
    nch = _SC_COLS // 128

    def _row_dma(j):
        return pltpu.async_copy(
            pred_hbm.at[base + j, pl.ds(_TC_COLS, _SC_COLS)],
            row_v.at[j % 2], row_sems[j % 2])

    acc = jnp.zeros((_L,), jnp.float32)
    lanes = lax.iota(jnp.int32, _L)
    pend = _row_dma(0)
    for j in range(bpw):
        nxt = _row_dma(j + 1) if j + 1 < bpw else None
        pend.wait()
        rbuf = row_v.at[j % 2]

        def _chunk_sum(k, a):
            b = k * 128
            for q in range(8):
                a = a + rbuf[pl.ds(b + q * _L, _L)]
            return a

        rsum16 = lax.fori_loop(0, nch, _chunk_sum, jnp.zeros((_L,), jnp.float32))
        t = tgt_v[pl.ds(j, _L)][0]
        mf = jnp.where(t > 0, jnp.float32(_EPS), jnp.float32(0.0))
        acc = acc + rsum16 * mf
        pend = nxt
    for c in copies:
        c.wait()
    for j in range(bpw):
        t = tgt_v[pl.ds(j, _L)][0]
        chunk = chunk_v[pl.ds(j * _L, _L)]
        lane = jnp.where(t > 0, t % _L, jnp.int32(-1))
        acc = acc + jnp.where(lanes == lane, chunk * jnp.float32(_CD),
                              jnp.float32(0.0))
    acc_v[...] = acc
    pltpu.sync_copy(acc_v, out_hbm.at[wid])


def kernel(prediction, target):
    rows = prediction.shape[0] * prediction.shape[1]
    vocab = prediction.shape[-1]
    pred = prediction.reshape(rows, vocab)
    tgt = target.reshape(rows, 1).astype(jnp.int32)
    tgt_flat = target.reshape(rows).astype(jnp.int32)
    bpw = rows // _NW

    mesh = plsc.VectorSubcoreMesh(core_axis_name="c", subcore_axis_name="s")
    sc_part = functools.partial(
        pl.kernel,
        mesh=mesh,
        out_type=jax.ShapeDtypeStruct((_NW, _L), jnp.float32),
        scratch_types=[
            pltpu.VMEM((bpw + _L,), jnp.int32),
            pltpu.VMEM((bpw * _L,), jnp.float32),
            pltpu.VMEM((2, _SC_COLS), jnp.float32),
            pltpu.VMEM((_L,), jnp.float32),
            pltpu.SemaphoreType.DMA,
            pltpu.SemaphoreType.DMA,
            pltpu.SemaphoreType.DMA,
        ],
    )(functools.partial(_sc_body, bpw))
    sc_parts = sc_part(pred, tgt_flat)

    col_block = 3200
    grid = (_TC_COLS // col_block,)
    s, n = pl.pallas_call(
        _tc_body,
        grid=grid,
        in_specs=[
            pl.BlockSpec((rows, col_block), lambda i: (0, i)),
            pl.BlockSpec((rows, 1), lambda i: (0, 0)),
        ],
        out_specs=[
            pl.BlockSpec((1, 1), lambda i: (0, 0)),
            pl.BlockSpec((1, 1), lambda i: (0, 0)),
        ],
        out_shape=[
            jax.ShapeDtypeStruct((1, 1), jnp.float32),
            jax.ShapeDtypeStruct((1, 1), jnp.float32),
        ],
    )(pred, tgt)

    total = _EPS * s[0, 0] + jnp.sum(sc_parts)
    return jnp.float32(_C) - total / n[0, 0]


# speedup vs baseline: 1.0204x; 1.0204x over previous
import functools
import math

import jax
import jax.numpy as jnp
from jax import lax
from jax.experimental import pallas as pl
from jax.experimental.pallas import tpu as pltpu
from jax.experimental.pallas import tpu_sc as plsc

_SMOOTH = 0.1
_VOCAB = 32000
_EPS = _SMOOTH / (_VOCAB - 1)
_CONF = 1.0 - _SMOOTH
_CD = _CONF - _EPS
_C = (_VOCAB - 1) * _EPS * math.log(_EPS) + _CONF * math.log(_CONF)

_NC, _NS, _L = 2, 16, 16
_NW = _NC * _NS


def _tc_body(pred_ref, tgt_ref, s_ref):
    i = pl.program_id(0)
    m = (tgt_ref[...] > 0).astype(jnp.float32)
    rs = jnp.sum(pred_ref[...], axis=1, keepdims=True)
    part = jnp.sum(rs * m)

    @pl.when(i == 0)
    def _():
        s_ref[...] = jnp.zeros_like(s_ref)

    s_ref[...] += part.reshape(1, 1)


def _sc_body(bpw, pred_hbm, tgt_hbm, out_hbm, tgt_v, chunk_v, acc_v, sem):
    wid = lax.axis_index("s") * _NC + lax.axis_index("c")
    base = wid * bpw
    pltpu.sync_copy(tgt_hbm.at[pl.ds(base, bpw)], tgt_v.at[pl.ds(0, bpw)])
    copies = []
    for j in range(bpw):
        t = tgt_v[pl.ds(j, _L)][0]
        r = base + j
        cc = (t // _L) * _L
        copies.append(pltpu.async_copy(
            pred_hbm.at[r, pl.ds(cc, _L)],
            chunk_v.at[pl.ds(j * _L, _L)], sem))
    for c in copies:
        c.wait()
    acc = jnp.zeros((_L,), jnp.float32)
    cnt = jnp.zeros((_L,), jnp.float32)
    lanes = lax.iota(jnp.int32, _L)
    for j in range(bpw):
        t = tgt_v[pl.ds(j, _L)][0]
        chunk = chunk_v[pl.ds(j * _L, _L)]
        lane = jnp.where(t > 0, t % _L, jnp.int32(-1))
        sel = lanes == lane
        acc = acc + jnp.where(sel, chunk, jnp.float32(0.0))
        cnt = cnt + jnp.where(sel, jnp.float32(1.0), jnp.float32(0.0))
    acc_v[pl.ds(0, _L)] = acc
    acc_v[pl.ds(_L, _L)] = cnt
    pltpu.sync_copy(acc_v, out_hbm.at[wid])


def kernel(prediction, target):
    rows = prediction.shape[0] * prediction.shape[1]
    vocab = prediction.shape[-1]
    pred = prediction.reshape(rows, vocab)
    tgt = target.reshape(rows, 1).astype(jnp.int32)
    tgt_flat = target.reshape(rows).astype(jnp.int32)
    bpw = rows // _NW

    mesh = plsc.VectorSubcoreMesh(core_axis_name="c", subcore_axis_name="s")
    sc_gather = functools.partial(
        pl.kernel,
        mesh=mesh,
        out_type=jax.ShapeDtypeStruct((_NW, 2 * _L), jnp.float32),
        scratch_types=[
            pltpu.VMEM((bpw + _L,), jnp.int32),
            pltpu.VMEM((bpw * _L,), jnp.float32),
            pltpu.VMEM((2 * _L,), jnp.float32),
            pltpu.SemaphoreType.DMA,
        ],
    )(functools.partial(_sc_body, bpw))
    g_parts = sc_gather(pred, tgt_flat)

    col_block = 3200
    grid = (vocab // col_block,)
    s = pl.pallas_call(
        _tc_body,
        grid=grid,
        in_specs=[
            pl.BlockSpec((rows, col_block), lambda i: (0, i)),
            pl.BlockSpec((rows, 1), lambda i: (0, 0)),
        ],
        out_specs=pl.BlockSpec((1, 1), lambda i: (0, 0)),
        out_shape=jax.ShapeDtypeStruct((1, 1), jnp.float32),
    )(pred, tgt)

    s2 = jnp.sum(g_parts[:, :_L])
    nval = jnp.sum(g_parts[:, _L:])
    total = _EPS * s[0, 0] + _CD * s2
    return jnp.float32(_C) - total / nval


# speedup vs baseline: 1.0214x; 1.0009x over previous
import functools
import math

import jax
import jax.numpy as jnp
from jax import lax
from jax.experimental import pallas as pl
from jax.experimental.pallas import tpu as pltpu
from jax.experimental.pallas import tpu_sc as plsc

_SMOOTH = 0.1
_VOCAB = 32000
_EPS = _SMOOTH / (_VOCAB - 1)
_CONF = 1.0 - _SMOOTH
_CD = _CONF - _EPS
_C = (_VOCAB - 1) * _EPS * math.log(_EPS) + _CONF * math.log(_CONF)

_NC, _NS, _L = 2, 16, 16
_NW = _NC * _NS


def _tc_body(pred_ref, tgt_ref, s_ref):
    i = pl.program_id(0)
    m = (tgt_ref[...] > 0).astype(jnp.float32)
    rs = jnp.sum(pred_ref[...], axis=1, keepdims=True)
    part = jnp.sum(rs * m)

    @pl.when(i == 0)
    def _():
        s_ref[...] = jnp.zeros_like(s_ref)

    s_ref[...] += part.reshape(1, 1)


def _sc_body(bpw, pred_hbm, tgt_hbm, out_hbm, tgt_v, chunk_v, acc_v, sem):
    wid = lax.axis_index("s") * _NC + lax.axis_index("c")
    base = wid * bpw
    pltpu.sync_copy(tgt_hbm.at[pl.ds(base, bpw)], tgt_v.at[pl.ds(0, bpw)])
    def _fire(j, carry):
        t = tgt_v[pl.ds(j, _L)][0]
        cc = (t // _L) * _L
        pltpu.async_copy(pred_hbm.at[base + j, pl.ds(cc, _L)],
                         chunk_v.at[pl.ds(j * _L, _L)], sem)
        return carry

    lax.fori_loop(0, bpw, _fire, 0)
    pltpu.make_async_copy(pred_hbm.at[0, pl.ds(0, bpw * _L)], chunk_v, sem).wait()

    lanes = lax.iota(jnp.int32, _L)

    def _accum(j, carry):
        acc, cnt = carry
        t = tgt_v[pl.ds(j, _L)][0]
        chunk = chunk_v[pl.ds(j * _L, _L)]
        lane = jnp.where(t > 0, t % _L, jnp.int32(-1))
        sel = lanes == lane
        acc = acc + jnp.where(sel, chunk, jnp.float32(0.0))
        cnt = cnt + jnp.where(sel, jnp.float32(1.0), jnp.float32(0.0))
        return acc, cnt

    acc, cnt = lax.fori_loop(
        0, bpw, _accum,
        (jnp.zeros((_L,), jnp.float32), jnp.zeros((_L,), jnp.float32)))
    acc_v[pl.ds(0, _L)] = acc
    acc_v[pl.ds(_L, _L)] = cnt
    pltpu.sync_copy(acc_v, out_hbm.at[wid])


def kernel(prediction, target):
    rows = prediction.shape[0] * prediction.shape[1]
    vocab = prediction.shape[-1]
    pred = prediction.reshape(rows, vocab)
    tgt = target.reshape(rows, 1).astype(jnp.int32)
    tgt_flat = target.reshape(rows).astype(jnp.int32)
    bpw = rows // _NW

    mesh = plsc.VectorSubcoreMesh(core_axis_name="c", subcore_axis_name="s")
    sc_gather = functools.partial(
        pl.kernel,
        mesh=mesh,
        out_type=jax.ShapeDtypeStruct((_NW, 2 * _L), jnp.float32),
        scratch_types=[
            pltpu.VMEM((bpw + _L,), jnp.int32),
            pltpu.VMEM((bpw * _L,), jnp.float32),
            pltpu.VMEM((2 * _L,), jnp.float32),
            pltpu.SemaphoreType.DMA,
        ],
    )(functools.partial(_sc_body, bpw))
    g_parts = sc_gather(pred, tgt_flat)

    col_block = 3200
    grid = (vocab // col_block,)
    s = pl.pallas_call(
        _tc_body,
        grid=grid,
        in_specs=[
            pl.BlockSpec((rows, col_block), lambda i: (0, i)),
            pl.BlockSpec((rows, 1), lambda i: (0, 0)),
        ],
        out_specs=pl.BlockSpec((1, 1), lambda i: (0, 0)),
        out_shape=jax.ShapeDtypeStruct((1, 1), jnp.float32),
    )(pred, tgt)

    s2 = jnp.sum(g_parts[:, :_L])
    nval = jnp.sum(g_parts[:, _L:])
    total = _EPS * s[0, 0] + _CD * s2
    return jnp.float32(_C) - total / nval


# speedup vs baseline: 1.0290x; 1.0075x over previous
import functools
import math

import jax
import jax.numpy as jnp
from jax import lax
from jax.experimental import pallas as pl
from jax.experimental.pallas import tpu as pltpu
from jax.experimental.pallas import tpu_sc as plsc

_SMOOTH = 0.1
_VOCAB = 32000
_EPS = _SMOOTH / (_VOCAB - 1)
_CONF = 1.0 - _SMOOTH
_CD = _CONF - _EPS
_C = (_VOCAB - 1) * _EPS * math.log(_EPS) + _CONF * math.log(_CONF)

_NC, _NS, _L = 2, 16, 16
_NW = _NC * _NS


def _tc_body(pred_ref, tgt_ref, s_ref):
    i = pl.program_id(0)
    m = (tgt_ref[...] > 0).astype(jnp.float32)
    rs = jnp.sum(pred_ref[...], axis=1, keepdims=True)
    part = jnp.sum(rs * m)

    @pl.when(i == 0)
    def _():
        s_ref[...] = jnp.zeros_like(s_ref)

    s_ref[...] += part.reshape(1, 1)


def _sc_body(bpw, pred_hbm, tgt_hbm, out_hbm, tgt_v, chunk_v, acc_v, sem):
    wid = lax.axis_index("s") * _NC + lax.axis_index("c")
    base = wid * bpw
    pltpu.sync_copy(tgt_hbm.at[pl.ds(base, bpw)], tgt_v.at[pl.ds(0, bpw)])
    def _fire(j, carry):
        t = tgt_v[pl.ds(j, _L)][0]
        cc = (t // _L) * _L
        pltpu.async_copy(pred_hbm.at[base + j, pl.ds(cc, _L)],
                         chunk_v.at[pl.ds(j * _L, _L)], sem)
        return carry

    lax.fori_loop(0, bpw, _fire, 0)
    pltpu.make_async_copy(pred_hbm.at[0, pl.ds(0, bpw * _L)], chunk_v, sem).wait()

    lanes = lax.iota(jnp.int32, _L)

    def _accum(j, carry):
        acc, cnt = carry
        t = tgt_v[pl.ds(j, _L)][0]
        chunk = chunk_v[pl.ds(j * _L, _L)]
        lane = jnp.where(t > 0, t % _L, jnp.int32(-1))
        sel = lanes == lane
        acc = acc + jnp.where(sel, chunk, jnp.float32(0.0))
        cnt = cnt + jnp.where(sel, jnp.float32(1.0), jnp.float32(0.0))
        return acc, cnt

    acc, cnt = lax.fori_loop(
        0, bpw, _accum,
        (jnp.zeros((_L,), jnp.float32), jnp.zeros((_L,), jnp.float32)))
    acc_v[pl.ds(0, _L)] = acc
    acc_v[pl.ds(_L, _L)] = cnt
    pltpu.sync_copy(acc_v, out_hbm.at[wid])


def kernel(prediction, target):
    rows = prediction.shape[0] * prediction.shape[1]
    vocab = prediction.shape[-1]
    pred = prediction.reshape(rows, vocab)
    tgt = target.reshape(rows, 1).astype(jnp.int32)
    tgt_flat = target.reshape(rows).astype(jnp.int32)
    bpw = rows // _NW

    mesh = plsc.VectorSubcoreMesh(core_axis_name="c", subcore_axis_name="s")
    sc_gather = functools.partial(
        pl.kernel,
        mesh=mesh,
        out_type=jax.ShapeDtypeStruct((_NW, 2 * _L), jnp.float32),
        scratch_types=[
            pltpu.VMEM((bpw + _L,), jnp.int32),
            pltpu.VMEM((bpw * _L,), jnp.float32),
            pltpu.VMEM((2 * _L,), jnp.float32),
            pltpu.SemaphoreType.DMA,
        ],
    )(functools.partial(_sc_body, bpw))
    g_parts = sc_gather(pred, tgt_flat)

    col_block = 1280
    grid = (vocab // col_block,)
    s = pl.pallas_call(
        _tc_body,
        grid=grid,
        in_specs=[
            pl.BlockSpec((rows, col_block), lambda i: (0, i)),
            pl.BlockSpec((rows, 1), lambda i: (0, 0)),
        ],
        out_specs=pl.BlockSpec((1, 1), lambda i: (0, 0)),
        out_shape=jax.ShapeDtypeStruct((1, 1), jnp.float32),
    )(pred, tgt)

    s2 = jnp.sum(g_parts[:, :_L])
    nval = jnp.sum(g_parts[:, _L:])
    total = _EPS * s[0, 0] + _CD * s2
    return jnp.float32(_C) - total / nval
